# baseline (speedup 1.0000x reference)
import functools
import math

import jax
import jax.numpy as jnp
from jax import lax
from jax.experimental import pallas as pl
from jax.experimental.pallas import tpu as pltpu
from jax.experimental.pallas import tpu_sc as plsc

D_MODEL = 64
SCALE = math.sqrt(D_MODEL)

NUM_CORES = 2
NUM_SUBCORES = 16
NUM_WORKERS = NUM_CORES * NUM_SUBCORES

BLK = 128


def _emb_body(xt_hbm, lut_hbm, out_hbm, idxblk, p0, p1, g0, g1, o0, o1,
              gsem, osem):
    wid = lax.axis_index("s") * NUM_CORES + lax.axis_index("c")
    n_seq = xt_hbm.shape[0]
    col0 = wid * BLK

    pidx = [p0, p1]
    grows = [g0, g1]
    ostage = [o0, o1]

    pltpu.sync_copy(xt_hbm.at[:, pl.ds(col0, BLK)], idxblk)

    def fill_pidx(s, sl):
        def vec(c, carry):
            v = idxblk[s, pl.ds(c * 16, 16)]
            pidx[sl][pl.ds(c * 16, 16)] = jax.lax.shift_right_logical(v, 1)
            return carry
        lax.fori_loop(0, BLK // 16, vec, 0, unroll=4)

    fill_pidx(0, 0)
    pltpu.async_copy(lut_hbm.at[pidx[0]], grows[0], gsem)
    fill_pidx(1, 1)
    pltpu.async_copy(lut_hbm.at[pidx[1]], grows[1], gsem)

    def step(s, sl):
        pltpu.make_async_copy(lut_hbm.at[pidx[sl]], grows[sl], gsem).wait()

        @pl.when(s >= 2)
        def _():
            pltpu.make_async_copy(
                ostage[sl], out_hbm.at[0, :, pl.ds(col0, BLK)], osem).wait()

        for c in range(BLK // 16):
            rowv = lax.iota(jnp.int32, 16) + (c * 16)
            offc = (idxblk[s, pl.ds(c * 16, 16)] & 1) * D_MODEL

            def dloop(d, carry):
                vals = plsc.load_gather(grows[sl], [rowv, offc + d])
                ostage[sl][d, pl.ds(c * 16, 16)] = vals * SCALE
                return carry
            lax.fori_loop(0, D_MODEL, dloop, 0, unroll=4)

        @pl.when(s + 2 < n_seq)
        def _():
            fill_pidx(s + 2, sl)
            pltpu.async_copy(lut_hbm.at[pidx[sl]], grows[sl], gsem)

        pltpu.async_copy(ostage[sl], out_hbm.at[s, :, pl.ds(col0, BLK)],
                         osem)

    def pair_body(i, carry):
        step(2 * i, 0)
        step(2 * i + 1, 1)
        return carry

    lax.fori_loop(0, n_seq // 2, pair_body, 0)

    for sl in range(2):
        pltpu.make_async_copy(
            ostage[sl], out_hbm.at[0, :, pl.ds(col0, BLK)], osem).wait()


def kernel(x, lut):
    b, s = x.shape
    xt = x.T.astype(jnp.int32)
    lut2 = lut.reshape(lut.shape[0] // 2, 2 * lut.shape[1])

    emb_call = pl.kernel(
        _emb_body,
        out_type=jax.ShapeDtypeStruct((s, D_MODEL, b), jnp.float32),
        mesh=plsc.VectorSubcoreMesh(
            core_axis_name="c", subcore_axis_name="s",
            num_cores=NUM_CORES, num_subcores=NUM_SUBCORES,
        ),
        scratch_types=[
            pltpu.VMEM((s, BLK), jnp.int32),
            pltpu.VMEM((BLK,), jnp.int32),
            pltpu.VMEM((BLK,), jnp.int32),
            pltpu.VMEM((BLK, 2 * D_MODEL), jnp.float32),
            pltpu.VMEM((BLK, 2 * D_MODEL), jnp.float32),
            pltpu.VMEM((D_MODEL, BLK), jnp.float32),
            pltpu.VMEM((D_MODEL, BLK), jnp.float32),
            pltpu.SemaphoreType.DMA,
            pltpu.SemaphoreType.DMA,
        ],
        compiler_params=pltpu.CompilerParams(
            use_tc_tiling_on_sc=True, needs_layout_passes=False),
    )
    out_t = emb_call(xt, lut2)
    return out_t.transpose(2, 0, 1)


# speedup vs baseline: 1.5615x; 1.5615x over previous
import functools
import math

import jax
import jax.numpy as jnp
from jax import lax
from jax.experimental import pallas as pl
from jax.experimental.pallas import tpu as pltpu
from jax.experimental.pallas import tpu_sc as plsc

D_MODEL = 64
SCALE = math.sqrt(D_MODEL)

NUM_CORES = 2
NUM_SUBCORES = 16
NUM_WORKERS = NUM_CORES * NUM_SUBCORES

BLK = 128


def _emb_body(xt_hbm, lut_hbm, out_hbm, idxblk, p0, p1, g0, g1, o0, o1,
              gsem, osem):
    wid = lax.axis_index("s") * NUM_CORES + lax.axis_index("c")
    n_seq = xt_hbm.shape[0]
    col0 = wid * BLK

    pidx = [p0, p1]
    grows = [g0, g1]
    ostage = [o0, o1]

    pltpu.sync_copy(xt_hbm.at[:, pl.ds(col0, BLK)], idxblk)

    def fill_pidx(s, sl):
        def vec(c, carry):
            v = idxblk[s, pl.ds(c * 16, 16)]
            pidx[sl][pl.ds(c * 16, 16)] = jax.lax.shift_right_logical(v, 1)
            return carry
        lax.fori_loop(0, BLK // 16, vec, 0, unroll=4)

    fill_pidx(0, 0)
    pltpu.async_copy(lut_hbm.at[pidx[0]], grows[0], gsem)
    fill_pidx(1, 1)
    pltpu.async_copy(lut_hbm.at[pidx[1]], grows[1], gsem)

    def step(s, sl):
        pltpu.make_async_copy(lut_hbm.at[pidx[sl]], grows[sl], gsem).wait()

        @pl.when(s >= 2)
        def _():
            pltpu.make_async_copy(
                ostage[sl], out_hbm.at[0, :, pl.ds(col0, BLK)], osem).wait()

        base = lax.iota(jnp.int32, 16)
        rowvs = [base + (c * 16) for c in range(BLK // 16)]
        offcs = [(idxblk[s, pl.ds(c * 16, 16)] & 1) * D_MODEL
                 for c in range(BLK // 16)]

        @plsc.parallel_loop(0, D_MODEL, unroll=4)
        def _(d):
            for c in range(BLK // 16):
                vals = plsc.load_gather(grows[sl], [rowvs[c], offcs[c] + d])
                ostage[sl][d, pl.ds(c * 16, 16)] = vals * SCALE

        @pl.when(s + 2 < n_seq)
        def _():
            fill_pidx(s + 2, sl)
            pltpu.async_copy(lut_hbm.at[pidx[sl]], grows[sl], gsem)

        pltpu.async_copy(ostage[sl], out_hbm.at[s, :, pl.ds(col0, BLK)],
                         osem)

    def pair_body(i, carry):
        step(2 * i, 0)
        step(2 * i + 1, 1)
        return carry

    lax.fori_loop(0, n_seq // 2, pair_body, 0)

    for sl in range(2):
        pltpu.make_async_copy(
            ostage[sl], out_hbm.at[0, :, pl.ds(col0, BLK)], osem).wait()


def kernel(x, lut):
    b, s = x.shape
    xt = x.T.astype(jnp.int32)
    lut2 = lut.reshape(lut.shape[0] // 2, 2 * lut.shape[1])

    emb_call = pl.kernel(
        _emb_body,
        out_type=jax.ShapeDtypeStruct((s, D_MODEL, b), jnp.float32),
        mesh=plsc.VectorSubcoreMesh(
            core_axis_name="c", subcore_axis_name="s",
            num_cores=NUM_CORES, num_subcores=NUM_SUBCORES,
        ),
        scratch_types=[
            pltpu.VMEM((s, BLK), jnp.int32),
            pltpu.VMEM((BLK,), jnp.int32),
            pltpu.VMEM((BLK,), jnp.int32),
            pltpu.VMEM((BLK, 2 * D_MODEL), jnp.float32),
            pltpu.VMEM((BLK, 2 * D_MODEL), jnp.float32),
            pltpu.VMEM((D_MODEL, BLK), jnp.float32),
            pltpu.VMEM((D_MODEL, BLK), jnp.float32),
            pltpu.SemaphoreType.DMA,
            pltpu.SemaphoreType.DMA,
        ],
        compiler_params=pltpu.CompilerParams(
            use_tc_tiling_on_sc=True, needs_layout_passes=False),
    )
    out_t = emb_call(xt, lut2)
    return out_t.transpose(2, 0, 1)
